# baseline (speedup 1.0000x reference)
import jax
import jax.numpy as jnp
from jax import lax
from jax.experimental import pallas as pl
from jax.experimental.pallas import tpu as pltpu
from jax.experimental.pallas import tpu_sc as plsc

V = 1000000
D = 64
B = 16384

NC = 2
NS = 16
L = 16
NW = NC * NS
BPW = B // NW
IC = 128
NCHUNK = BPW // IC
GPC = IC // L
DCH = D // L
W2 = 2 * D


def _glove_body(wi_hbm, wj_hbm, bi_hbm, bj_hbm, idx_hbm, off_hbm,
                out_hbm, idx_v, off_v, rows_i, rows_j, bi_v, bj_v,
                out_v, sem0, sem1, bsem):
    wid = lax.axis_index("s") * NC + lax.axis_index("c")
    base = wid * BPW

    pltpu.sync_copy(idx_hbm.at[wid], idx_v)
    pltpu.sync_copy(off_hbm.at[wid], off_v)

    bias_copies = []
    for c in range(NCHUNK):
        sl = pl.ds(c * IC, IC)
        bias_copies.append(pltpu.async_copy(bi_hbm.at[idx_v.at[2, c]],
                                            bi_v.at[sl], bsem))
        bias_copies.append(pltpu.async_copy(bj_hbm.at[idx_v.at[3, c]],
                                            bj_v.at[sl], bsem))

    def fire(c, buf):
        sem = sem0 if buf == 0 else sem1
        return (pltpu.async_copy(wi_hbm.at[idx_v.at[0, c]],
                                 rows_i.at[buf], sem),
                pltpu.async_copy(wj_hbm.at[idx_v.at[1, c]],
                                 rows_j.at[buf], sem))

    lane = lax.iota(jnp.int32, L)
    dnums = lax.GatherDimensionNumbers(
        offset_dims=(), collapsed_slice_dims=(0,), start_index_map=(0,))

    def vperm(v, idx):
        return lax.gather(v, idx[:, None], dnums, slice_sizes=(1,),
                          mode=lax.GatherScatterMode.PROMISE_IN_BOUNDS)

    inflight = fire(0, 0)
    for c in range(NCHUNK):
        inflight[0].wait()
        inflight[1].wait()
        if c + 1 < NCHUNK:
            nxt = fire(c + 1, (c + 1) % 2)
        buf = c % 2

        def group(g, carry):
            row0 = c * IC + g * L
            ov_i = off_v[0, pl.ds(row0, L)]
            ov_j = off_v[1, pl.ds(row0, L)]
            out16 = jnp.zeros((L,), jnp.float32)
            for r in range(L):
                rowl = g * L + r
                oi = ov_i[r]
                oj = ov_j[r]
                acc = None
                for cch in range(DCH):
                    a = rows_i[buf, rowl, pl.ds(oi + cch * L, L)]
                    b = rows_j[buf, rowl, pl.ds(oj + cch * L, L)]
                    acc = a * b if acc is None else acc + a * b
                for sh in (8, 4, 2, 1):
                    acc = acc + vperm(acc, lane ^ sh)
                out16 = jnp.where(lane == r, acc, out16)
            out_v[pl.ds(row0, L)] = out16
            return carry

        lax.fori_loop(0, GPC, group, 0)
        if c + 1 < NCHUNK:
            inflight = nxt

    for cp in bias_copies:
        cp.wait()

    def biasadd(g, carry):
        sl = pl.ds(g * L, L)
        out_v[sl] = out_v[sl] + bi_v[sl] + bj_v[sl]
        return carry

    lax.fori_loop(0, NCHUNK * GPC, biasadd, 0)

    pltpu.sync_copy(out_v, out_hbm.at[pl.ds(base, BPW)])


@jax.jit
def _glove(idx2, off2, wi2, wj2, bi1, bj1):
    mesh = plsc.VectorSubcoreMesh(core_axis_name="c", subcore_axis_name="s")
    run = pl.kernel(
        _glove_body,
        out_type=jax.ShapeDtypeStruct((B,), jnp.float32),
        mesh=mesh,
        scratch_types=[
            pltpu.VMEM((4, NCHUNK, IC), jnp.int32),
            pltpu.VMEM((2, BPW), jnp.int32),
            pltpu.VMEM((2, IC, W2), jnp.float32),
            pltpu.VMEM((2, IC, W2), jnp.float32),
            pltpu.VMEM((BPW,), jnp.float32),
            pltpu.VMEM((BPW,), jnp.float32),
            pltpu.VMEM((BPW,), jnp.float32),
            pltpu.SemaphoreType.DMA,
            pltpu.SemaphoreType.DMA,
            pltpu.SemaphoreType.DMA,
        ],
    )
    return run(wi2, wj2, bi1, bj1, idx2, off2)


def kernel(word_i, word_j, wi, wj, bi, bj):
    wi32 = word_i.astype(jnp.int32)
    wj32 = word_j.astype(jnp.int32)
    idx2 = jnp.stack([wi32 >> 1, wj32 >> 1, wi32, wj32])
    idx2 = idx2.reshape(4, NW, NCHUNK, IC)
    idx2 = idx2.transpose(1, 0, 2, 3)
    off2 = jnp.stack([(wi32 & 1) * D, (wj32 & 1) * D]).reshape(2, NW, BPW)
    off2 = off2.transpose(1, 0, 2)
    wi2 = wi.reshape(V // 2, W2)
    wj2 = wj.reshape(V // 2, W2)
    return _glove(idx2, off2, wi2, wj2, bi.reshape(V), bj.reshape(V))
